# baseline (speedup 1.0000x reference)
import jax
import jax.numpy as jnp
from jax.experimental import pallas as pl
from jax.experimental.pallas import tpu as pltpu

_B, _H, _S, _D = 8, 16, 4096, 128
_Q = 16
_BH = _B * _H
_BB = 4
_NBUF = 3
_NSTEP = _BH // _BB


def _body(pos_ref, kc_ref, vc_ref, kv_ref, vv_ref, ko_ref, vo_ref,
          kbuf, vbuf, in_sem, out_sem):
    i = pl.program_id(0)
    slot = jax.lax.rem(i, _NBUF)

    def in_cp(step, slot_, c_ref, buf, cidx):
        return pltpu.make_async_copy(
            c_ref.at[pl.ds(step * _BB, _BB)], buf.at[slot_],
            in_sem.at[slot_, cidx])

    def out_cp(step, slot_, buf, o_ref, cidx):
        return pltpu.make_async_copy(
            buf.at[slot_], o_ref.at[pl.ds(step * _BB, _BB)],
            out_sem.at[slot_, cidx])

    @pl.when(i == 0)
    def _():
        for j in range(_NBUF):
            in_cp(j, j, kc_ref, kbuf, 0).start()
            in_cp(j, j, vc_ref, vbuf, 1).start()

    in_cp(i, slot, kc_ref, kbuf, 0).wait()
    in_cp(i, slot, vc_ref, vbuf, 1).wait()

    row_ids = jax.lax.broadcasted_iota(jnp.int32, (_BB, 8, _D), 1)
    for q in range(_Q):
        p = pos_ref[q]
        base = (p // 8) * 8
        sel = row_ids == (p - base)
        for val_ref, buf in ((kv_ref, kbuf), (vv_ref, vbuf)):
            row = jnp.broadcast_to(
                val_ref[pl.ds(i * _BB, _BB), pl.ds(q, 1), :], (_BB, 8, _D))
            chunk = buf[slot, :, pl.ds(base, 8), :]
            buf[slot, :, pl.ds(base, 8), :] = jnp.where(sel, row, chunk)

    out_cp(i, slot, kbuf, ko_ref, 0).start()
    out_cp(i, slot, vbuf, vo_ref, 1).start()

    nslot = jax.lax.rem(i + _NBUF - 1, _NBUF)

    @pl.when((i >= 1) & (i + _NBUF - 1 < _NSTEP))
    def _():
        out_cp(i - 1, nslot, kbuf, ko_ref, 0).wait()
        out_cp(i - 1, nslot, vbuf, vo_ref, 1).wait()
        in_cp(i + _NBUF - 1, nslot, kc_ref, kbuf, 0).start()
        in_cp(i + _NBUF - 1, nslot, vc_ref, vbuf, 1).start()

    @pl.when(i == _NSTEP - 1)
    def _():
        for s in range(_NSTEP - _NBUF, _NSTEP):
            out_cp(s, s % _NBUF, kbuf, ko_ref, 0).wait()
            out_cp(s, s % _NBUF, vbuf, vo_ref, 1).wait()


def kernel(input_pos, k_val, v_val, k_cache, v_cache):
    kc = k_cache.reshape(_BH, _S, _D)
    vc = v_cache.reshape(_BH, _S, _D)
    kv = k_val.reshape(_BH, _Q, _D)
    vv = v_val.reshape(_BH, _Q, _D)
    grid_spec = pltpu.PrefetchScalarGridSpec(
        num_scalar_prefetch=1,
        grid=(_NSTEP,),
        in_specs=[
            pl.BlockSpec(memory_space=pltpu.MemorySpace.HBM),
            pl.BlockSpec(memory_space=pltpu.MemorySpace.HBM),
            pl.BlockSpec((_BH, _Q, _D), lambda i, pos: (0, 0, 0)),
            pl.BlockSpec((_BH, _Q, _D), lambda i, pos: (0, 0, 0)),
        ],
        out_specs=[
            pl.BlockSpec(memory_space=pltpu.MemorySpace.HBM),
            pl.BlockSpec(memory_space=pltpu.MemorySpace.HBM),
        ],
        scratch_shapes=[
            pltpu.VMEM((_NBUF, _BB, _S, _D), jnp.bfloat16),
            pltpu.VMEM((_NBUF, _BB, _S, _D), jnp.bfloat16),
            pltpu.SemaphoreType.DMA((_NBUF, 2)),
            pltpu.SemaphoreType.DMA((_NBUF, 2)),
        ],
    )
    ko, vo = pl.pallas_call(
        _body,
        grid_spec=grid_spec,
        out_shape=[
            jax.ShapeDtypeStruct((_BH, _S, _D), k_cache.dtype),
            jax.ShapeDtypeStruct((_BH, _S, _D), v_cache.dtype),
        ],
    )(input_pos, kc, vc, kv, vv)
    return ko.reshape(_B, _H, _S, _D), vo.reshape(_B, _H, _S, _D)
